# baseline (speedup 1.0000x reference)
import functools

import jax
import jax.numpy as jnp
from jax.experimental import pallas as pl
from jax.experimental.pallas import tpu as pltpu

_INTER = 512
_E = 8
_HALF_H = 512


def _moe_body(x_hbm, gu_hbm, dn_hbm, idx_ref, wts_ref, out_ref,
              x_buf, gu_buf, dn_buf, x_sem, gu_sem_a, gu_sem_b,
              dn_sem_a, dn_sem_b):
    def gu_copy_a(e):
        return pltpu.make_async_copy(
            gu_hbm.at[e, :_INTER], gu_buf.at[e % 2, :_INTER],
            gu_sem_a.at[e % 2])

    def gu_copy_b(e):
        return pltpu.make_async_copy(
            gu_hbm.at[e, _INTER:], gu_buf.at[e % 2, _INTER:],
            gu_sem_b.at[e % 2])

    def dn_copy_a(e):
        return pltpu.make_async_copy(
            dn_hbm.at[e, :_HALF_H], dn_buf.at[e % 2, :_HALF_H],
            dn_sem_a.at[e % 2])

    def dn_copy_b(e):
        return pltpu.make_async_copy(
            dn_hbm.at[e, _HALF_H:], dn_buf.at[e % 2, _HALF_H:],
            dn_sem_b.at[e % 2])

    x_copy = pltpu.make_async_copy(x_hbm, x_buf, x_sem)

    gu_copy_a(0).start()
    x_copy.start()
    gu_copy_b(0).start()
    dn_copy_a(0).start()
    dn_copy_b(0).start()
    x_copy.wait()
    x = x_buf[...]
    for e in range(_E):
        b = e % 2
        if e + 1 < _E:
            gu_copy_a(e + 1).start()
            gu_copy_b(e + 1).start()
            dn_copy_a(e + 1).start()
            dn_copy_b(e + 1).start()
        gu_copy_a(e).wait()
        gate = jax.lax.dot_general(
            x, gu_buf[b, :_INTER], (((1,), (1,)), ((), ())),
            preferred_element_type=jnp.float32)
        gu_copy_b(e).wait()
        up = jax.lax.dot_general(
            x, gu_buf[b, _INTER:], (((1,), (1,)), ((), ())),
            preferred_element_type=jnp.float32)
        h = gate * jax.nn.sigmoid(gate) * up
        sel = (idx_ref[...] == e).astype(jnp.float32)
        w = jnp.sum(wts_ref[...] * sel, axis=1, keepdims=True)
        dn_copy_a(e).wait()
        ca = jax.lax.dot_general(
            h, dn_buf[b, :_HALF_H], (((1,), (1,)), ((), ())),
            preferred_element_type=jnp.float32) * w
        if e == 0:
            out_ref[:, :_HALF_H] = ca
        else:
            out_ref[:, :_HALF_H] += ca
        dn_copy_b(e).wait()
        cb = jax.lax.dot_general(
            h, dn_buf[b, _HALF_H:], (((1,), (1,)), ((), ())),
            preferred_element_type=jnp.float32) * w
        if e == 0:
            out_ref[:, _HALF_H:] = cb
        else:
            out_ref[:, _HALF_H:] += cb


@jax.jit
def kernel(hidden_states, top_k_index, top_k_weights, gate_up_proj, down_proj):
    n, h = hidden_states.shape
    e = gate_up_proj.shape[0]
    i2 = gate_up_proj.shape[1]
    i = down_proj.shape[2]
    out = pl.pallas_call(
        _moe_body,
        in_specs=[
            pl.BlockSpec(memory_space=pltpu.MemorySpace.HBM),
            pl.BlockSpec(memory_space=pltpu.MemorySpace.HBM),
            pl.BlockSpec(memory_space=pltpu.MemorySpace.HBM),
            pl.BlockSpec(memory_space=pltpu.MemorySpace.VMEM),
            pl.BlockSpec(memory_space=pltpu.MemorySpace.VMEM),
        ],
        out_specs=pl.BlockSpec(memory_space=pltpu.MemorySpace.VMEM),
        out_shape=jax.ShapeDtypeStruct((n, h), jnp.float32),
        scratch_shapes=[
            pltpu.VMEM((n, h), jnp.float32),
            pltpu.VMEM((2, i2, h), jnp.float32),
            pltpu.VMEM((2, h, i), jnp.float32),
            pltpu.SemaphoreType.DMA,
            pltpu.SemaphoreType.DMA((2,)),
            pltpu.SemaphoreType.DMA((2,)),
            pltpu.SemaphoreType.DMA((2,)),
            pltpu.SemaphoreType.DMA((2,)),
        ],
    )(hidden_states, gate_up_proj, down_proj,
      top_k_index.astype(jnp.int32), top_k_weights)
    return out.astype(hidden_states.dtype)


# speedup vs baseline: 1.0229x; 1.0229x over previous
import functools

import jax
import jax.numpy as jnp
from jax.experimental import pallas as pl
from jax.experimental.pallas import tpu as pltpu

_INTER = 512
_E = 8
_HALF_H = 512


def _moe_body(x_ref, gu_hbm, dn_hbm, idx_ref, wts_ref, out_ref,
              gu_buf, dn_buf, gu_sem_a, gu_sem_b, dn_sem_a, dn_sem_b):
    def gu_copy_a(e):
        return pltpu.make_async_copy(
            gu_hbm.at[e, :_INTER], gu_buf.at[e % 2, :_INTER],
            gu_sem_a.at[e % 2])

    def gu_copy_b(e):
        return pltpu.make_async_copy(
            gu_hbm.at[e, _INTER:], gu_buf.at[e % 2, _INTER:],
            gu_sem_b.at[e % 2])

    def dn_copy_a(e):
        return pltpu.make_async_copy(
            dn_hbm.at[e, :_HALF_H], dn_buf.at[e % 2, :_HALF_H],
            dn_sem_a.at[e % 2])

    def dn_copy_b(e):
        return pltpu.make_async_copy(
            dn_hbm.at[e, _HALF_H:], dn_buf.at[e % 2, _HALF_H:],
            dn_sem_b.at[e % 2])

    def start_all(e):
        gu_copy_a(e).start()
        gu_copy_b(e).start()
        dn_copy_a(e).start()
        dn_copy_b(e).start()

    start_all(0)
    x = x_ref[...]
    for e in range(_E):
        b = e % 2
        if e + 1 < _E:
            start_all(e + 1)
        gu_copy_a(e).wait()
        gu_copy_b(e).wait()
        proj = jax.lax.dot_general(
            x, gu_buf[b], (((1,), (1,)), ((), ())),
            preferred_element_type=jnp.float32)
        gate = proj[:, :_INTER]
        up = proj[:, _INTER:]
        h = gate * jax.nn.sigmoid(gate) * up
        dn_copy_a(e).wait()
        dn_copy_b(e).wait()
        out_e = jax.lax.dot_general(
            h, dn_buf[b], (((1,), (1,)), ((), ())),
            preferred_element_type=jnp.float32)
        sel = (idx_ref[...] == e).astype(jnp.float32)
        w = jnp.sum(wts_ref[...] * sel, axis=1, keepdims=True)
        contrib = out_e * w
        if e == 0:
            out_ref[...] = contrib
        else:
            out_ref[...] += contrib


@jax.jit
def kernel(hidden_states, top_k_index, top_k_weights, gate_up_proj, down_proj):
    n, h = hidden_states.shape
    e = gate_up_proj.shape[0]
    i2 = gate_up_proj.shape[1]
    i = down_proj.shape[2]
    out = pl.pallas_call(
        _moe_body,
        in_specs=[
            pl.BlockSpec(memory_space=pltpu.MemorySpace.VMEM),
            pl.BlockSpec(memory_space=pltpu.MemorySpace.HBM),
            pl.BlockSpec(memory_space=pltpu.MemorySpace.HBM),
            pl.BlockSpec(memory_space=pltpu.MemorySpace.VMEM),
            pl.BlockSpec(memory_space=pltpu.MemorySpace.VMEM),
        ],
        out_specs=pl.BlockSpec(memory_space=pltpu.MemorySpace.VMEM),
        out_shape=jax.ShapeDtypeStruct((n, h), jnp.float32),
        scratch_shapes=[
            pltpu.VMEM((2, i2, h), jnp.float32),
            pltpu.VMEM((2, h, i), jnp.float32),
            pltpu.SemaphoreType.DMA((2,)),
            pltpu.SemaphoreType.DMA((2,)),
            pltpu.SemaphoreType.DMA((2,)),
            pltpu.SemaphoreType.DMA((2,)),
        ],
    )(hidden_states, gate_up_proj, down_proj,
      top_k_index.astype(jnp.int32), top_k_weights)
    return out.astype(hidden_states.dtype)
